# baseline (speedup 1.0000x reference)
import functools

import jax
import jax.numpy as jnp
from jax import lax
from jax.experimental import pallas as pl
from jax.experimental.pallas import tpu as pltpu
from jax.experimental.pallas import tpu_sc as plsc

NC = 2
NS = 16
NW = NC * NS
B = 128


def _cdiv(a, b):
    return (a + b - 1) // b


def _chunks(total, step):
    out = []
    off = 0
    while off < total:
        out.append((off, min(step, total - off)))
        off += step
    return out


@functools.lru_cache(maxsize=None)
def _make_degree_kernel(n, kd, nacc, rpt):
    mesh = plsc.VectorSubcoreMesh(core_axis_name="c", subcore_axis_name="s")

    @functools.partial(
        pl.kernel,
        out_type=jax.ShapeDtypeStruct((NC * n,), jnp.float32),
        mesh=mesh,
        scratch_types=[
            pltpu.VMEM((kd, B), jnp.int32),
            pltpu.VMEM((kd, B), jnp.float32),
            pltpu.VMEM((rpt,), jnp.float32),
            pltpu.VMEM_SHARED((nacc,), jnp.float32),
        ],
    )
    def deg_kernel(src_hbm, dst_hbm, vals_hbm, zz_hbm, out_hbm, idx_v, val_v,
                   zbuf_v, acc):
        cid = lax.axis_index("c")
        sid = lax.axis_index("s")
        pltpu.sync_copy(zz_hbm, zbuf_v)
        pltpu.sync_copy(zbuf_v, acc.at[pl.ds(sid * rpt, rpt)])

        @pl.when(cid == 0)
        def _():
            pltpu.sync_copy(src_hbm.at[pl.ds(sid * kd, kd)], idx_v)

        @pl.when(cid == 1)
        def _():
            pltpu.sync_copy(dst_hbm.at[pl.ds(sid * kd, kd)], idx_v)

        pltpu.sync_copy(vals_hbm.at[pl.ds(sid * kd, kd)], val_v)
        plsc.subcore_barrier()

        def body(j, c):
            pltpu.sync_copy(val_v.at[j], acc.at[idx_v.at[j]], add=True)
            return c

        lax.fori_loop(0, kd, body, 0)
        plsc.subcore_barrier()

        full_tiles = n // rpt
        rem = n - full_tiles * rpt

        @pl.when(sid < full_tiles)
        def _():
            pltpu.sync_copy(acc.at[pl.ds(sid * rpt, rpt)], zbuf_v)
            pltpu.sync_copy(zbuf_v, out_hbm.at[pl.ds(cid * n + sid * rpt, rpt)])

        if rem:
            @pl.when(sid == full_tiles)
            def _():
                pltpu.sync_copy(acc.at[pl.ds(sid * rpt, rem)],
                                zbuf_v.at[pl.ds(0, rem)])
                pltpu.sync_copy(zbuf_v.at[pl.ds(0, rem)],
                                out_hbm.at[pl.ds(cid * n + sid * rpt, rem)])

    return deg_kernel


@functools.lru_cache(maxsize=None)
def _make_prop_kernel(n, d, k1, nacc, rpt):
    mesh = plsc.VectorSubcoreMesh(core_axis_name="c", subcore_axis_name="s")
    grp = 40
    while k1 % grp or grp % 2:
        grp -= 8
    npairs = grp // 2
    ngrp = k1 // grp

    @functools.partial(
        pl.kernel,
        out_type=jax.ShapeDtypeStruct((NC, n, d), jnp.float32),
        mesh=mesh,
        scratch_types=[
            pltpu.VMEM((grp, B), jnp.int32),
            pltpu.VMEM((grp, B), jnp.int32),
            pltpu.VMEM((B, d), jnp.float32),
            pltpu.VMEM((B, d), jnp.float32),
            pltpu.SemaphoreType.DMA,
            pltpu.SemaphoreType.DMA,
            pltpu.SemaphoreType.DMA,
            pltpu.SemaphoreType.DMA,
            pltpu.VMEM_SHARED((nacc, d), jnp.float32),
        ],
    )
    def prop_kernel(x_hbm, src_hbm, dst_hbm, zz_hbm, out_hbm,
                    sidx_v, didx_v, rows_a, rows_b, sga, sgb, ssa, ssb, acc):
        cid = lax.axis_index("c")
        sid = lax.axis_index("s")
        wid = cid * NS + sid
        pltpu.sync_copy(zz_hbm, rows_a)
        for off, sz in _chunks(rpt, B):
            pltpu.sync_copy(rows_a.at[pl.ds(0, sz), :],
                            acc.at[pl.ds(sid * rpt + off, sz), :])
        plsc.subcore_barrier()
        base = wid * k1

        for g in range(ngrp):
            pltpu.sync_copy(src_hbm.at[pl.ds(base + g * grp, grp)], sidx_v)
            pltpu.sync_copy(dst_hbm.at[pl.ds(base + g * grp, grp)], didx_v)
            pltpu.async_copy(x_hbm.at[sidx_v.at[0]], rows_a, sga)

            def body(i, c):
                @pl.when(i > 0)
                def _():
                    pltpu.make_async_copy(
                        rows_b, acc.at[didx_v.at[2 * i - 1]], ssb).wait()

                pltpu.async_copy(x_hbm.at[sidx_v.at[2 * i + 1]], rows_b, sgb)
                pltpu.make_async_copy(
                    x_hbm.at[sidx_v.at[2 * i]], rows_a, sga).wait()
                pltpu.async_copy(rows_a, acc.at[didx_v.at[2 * i]], ssa,
                                 add=True)
                pltpu.make_async_copy(
                    rows_a, acc.at[didx_v.at[2 * i]], ssa).wait()

                @pl.when(i < npairs - 1)
                def _():
                    pltpu.async_copy(x_hbm.at[sidx_v.at[2 * i + 2]], rows_a,
                                     sga)

                pltpu.make_async_copy(
                    x_hbm.at[sidx_v.at[2 * i + 1]], rows_b, sgb).wait()
                pltpu.async_copy(rows_b, acc.at[didx_v.at[2 * i + 1]], ssb,
                                 add=True)
                return c

            lax.fori_loop(0, npairs, body, 0)
            pltpu.make_async_copy(rows_b, acc.at[didx_v.at[grp - 1]],
                                  ssb).wait()
        plsc.subcore_barrier()

        full_tiles = n // rpt
        rem = n - full_tiles * rpt

        @pl.when(sid < full_tiles)
        def _():
            for off, sz in _chunks(rpt, B):
                pltpu.sync_copy(acc.at[pl.ds(sid * rpt + off, sz), :],
                                rows_a.at[pl.ds(0, sz), :])
                pltpu.sync_copy(rows_a.at[pl.ds(0, sz), :],
                                out_hbm.at[cid, pl.ds(sid * rpt + off, sz), :])

        if rem:
            @pl.when(sid == full_tiles)
            def _():
                for off, sz in _chunks(rem, B):
                    pltpu.sync_copy(acc.at[pl.ds(sid * rpt + off, sz), :],
                                    rows_a.at[pl.ds(0, sz), :])
                    pltpu.sync_copy(rows_a.at[pl.ds(0, sz), :],
                                    out_hbm.at[cid, pl.ds(sid * rpt + off, sz), :])

    return prop_kernel


def _mm1_body(x_ref, dego_ref, w_ref, o_ref):
    no = lax.rsqrt(jnp.maximum(dego_ref[...], 1.0))
    o_ref[...] = jnp.dot(x_ref[...] * no, w_ref[...],
                         preferred_element_type=jnp.float32)


def _mm2_body(p0_ref, p1_ref, degi_ref, dego_ref, b1_ref, w_ref, o_ref):
    ni = lax.rsqrt(jnp.maximum(degi_ref[...], 1.0))
    no = lax.rsqrt(jnp.maximum(dego_ref[...], 1.0))
    h = jax.nn.relu((p0_ref[...] + p1_ref[...]) * ni + b1_ref[...])
    x2 = jnp.dot(h * no, w_ref[...], preferred_element_type=jnp.float32)
    bn, ncls = x2.shape
    pad_w = o_ref.shape[1] - ncls
    o_ref[...] = jnp.concatenate(
        [x2, jnp.zeros((bn, pad_w), jnp.float32)], axis=1)


def _fin_body(q0_ref, q1_ref, degi_ref, b2_ref, o_ref):
    ni = lax.rsqrt(jnp.maximum(degi_ref[...], 1.0))
    ncls = o_ref.shape[1]
    s = q0_ref[...] + q1_ref[...]
    o_ref[...] = s[:, :ncls] * ni + b2_ref[...]


def _row_block(bn, bd):
    return pl.BlockSpec((bn, bd), lambda i: (i, 0))


def _full_block(shape):
    return pl.BlockSpec(shape, lambda i: tuple(0 for _ in shape))


def kernel(in_feat, edge_index, W1, b1, W2, b2):
    n, d_in = in_feat.shape
    d_h = W1.shape[1]
    n_cls = W2.shape[1]
    e = edge_index.shape[1]

    k1 = ((_cdiv(e, NW * B) + 7) // 8) * 8
    e_pad = NW * B * k1
    kd = 2 * k1
    rpt = ((_cdiv(n + 1, NS) + 7) // 8) * 8
    nacc = NS * rpt
    assert n % 1000 == 0

    pad = e_pad - e
    src = edge_index[0]
    dst = edge_index[1]
    src2d = jnp.concatenate([src, jnp.zeros((pad,), jnp.int32)]).reshape(-1, B)
    dummy = n + (jnp.arange(pad, dtype=jnp.int32) % 96)
    dst2d = jnp.concatenate([dst, dummy]).reshape(-1, B)
    ev2d = jnp.concatenate([jnp.ones((e,), jnp.float32),
                            jnp.zeros((pad,), jnp.float32)]).reshape(-1, B)
    zd = jnp.zeros((rpt,), jnp.float32)
    zp1 = jnp.zeros((B, d_h), jnp.float32)

    deg = _make_degree_kernel(n, kd, nacc, rpt)(src2d, dst2d, ev2d, zd)
    dego = deg[:n].reshape(n, 1)
    degi = deg[n:].reshape(n, 1)

    bn = 1000
    grid = (n // bn,)

    x1 = pl.pallas_call(
        _mm1_body,
        grid=grid,
        in_specs=[_row_block(bn, d_in), _row_block(bn, 1),
                  _full_block((d_in, d_h))],
        out_specs=_row_block(bn, d_h),
        out_shape=jax.ShapeDtypeStruct((n, d_h), jnp.float32),
    )(in_feat, dego, W1)

    parts1 = _make_prop_kernel(n, d_h, k1, nacc, rpt)(x1, src2d, dst2d, zp1)

    x2 = pl.pallas_call(
        _mm2_body,
        grid=grid,
        in_specs=[_row_block(bn, d_h), _row_block(bn, d_h),
                  _row_block(bn, 1), _row_block(bn, 1),
                  _full_block((1, d_h)), _full_block((d_h, n_cls))],
        out_specs=_row_block(bn, d_h),
        out_shape=jax.ShapeDtypeStruct((n, d_h), jnp.float32),
    )(parts1[0], parts1[1], degi, dego, b1.reshape(1, d_h), W2)

    parts2 = _make_prop_kernel(n, d_h, k1, nacc, rpt)(x2, src2d, dst2d, zp1)

    out = pl.pallas_call(
        _fin_body,
        grid=grid,
        in_specs=[_row_block(bn, d_h), _row_block(bn, d_h),
                  _row_block(bn, 1), _full_block((1, n_cls))],
        out_specs=_row_block(bn, n_cls),
        out_shape=jax.ShapeDtypeStruct((n, n_cls), jnp.float32),
    )(parts2[0], parts2[1], degi, b2.reshape(1, n_cls))

    return out


# speedup vs baseline: 1.1900x; 1.1900x over previous
import functools

import jax
import jax.numpy as jnp
from jax import lax
from jax.experimental import pallas as pl
from jax.experimental.pallas import tpu as pltpu
from jax.experimental.pallas import tpu_sc as plsc

NC = 2
NS = 16
NW = NC * NS
B = 128


def _cdiv(a, b):
    return (a + b - 1) // b


def _chunks(total, step):
    out = []
    off = 0
    while off < total:
        out.append((off, min(step, total - off)))
        off += step
    return out


@functools.lru_cache(maxsize=None)
def _make_degree_kernel(n, kd, nacc, rpt):
    mesh = plsc.VectorSubcoreMesh(core_axis_name="c", subcore_axis_name="s")

    @functools.partial(
        pl.kernel,
        out_type=jax.ShapeDtypeStruct((NC * n,), jnp.float32),
        mesh=mesh,
        scratch_types=[
            pltpu.VMEM((kd, B), jnp.int32),
            pltpu.VMEM((kd, B), jnp.float32),
            pltpu.VMEM((rpt,), jnp.float32),
            pltpu.VMEM_SHARED((nacc,), jnp.float32),
        ],
        compiler_params=pltpu.CompilerParams(use_tc_tiling_on_sc=False),
    )
    def deg_kernel(src_hbm, dst_hbm, vals_hbm, zz_hbm, out_hbm, idx_v, val_v,
                   zbuf_v, acc):
        cid = lax.axis_index("c")
        sid = lax.axis_index("s")
        pltpu.sync_copy(zz_hbm, zbuf_v)
        pltpu.sync_copy(zbuf_v, acc.at[pl.ds(sid * rpt, rpt)])

        @pl.when(cid == 0)
        def _():
            pltpu.sync_copy(src_hbm.at[pl.ds(sid * kd, kd)], idx_v)

        @pl.when(cid == 1)
        def _():
            pltpu.sync_copy(dst_hbm.at[pl.ds(sid * kd, kd)], idx_v)

        pltpu.sync_copy(vals_hbm.at[pl.ds(sid * kd, kd)], val_v)
        plsc.subcore_barrier()

        def body(j, c):
            pltpu.sync_copy(val_v.at[j], acc.at[idx_v.at[j]], add=True)
            return c

        lax.fori_loop(0, kd, body, 0)
        plsc.subcore_barrier()

        full_tiles = n // rpt
        rem = n - full_tiles * rpt

        @pl.when(sid < full_tiles)
        def _():
            pltpu.sync_copy(acc.at[pl.ds(sid * rpt, rpt)], zbuf_v)
            pltpu.sync_copy(zbuf_v, out_hbm.at[pl.ds(cid * n + sid * rpt, rpt)])

        if rem:
            @pl.when(sid == full_tiles)
            def _():
                pltpu.sync_copy(acc.at[pl.ds(sid * rpt, rem)],
                                zbuf_v.at[pl.ds(0, rem)])
                pltpu.sync_copy(zbuf_v.at[pl.ds(0, rem)],
                                out_hbm.at[pl.ds(cid * n + sid * rpt, rem)])

    return deg_kernel


@functools.lru_cache(maxsize=None)
def _make_prop_kernel(n, d, k1, nacc, rpt):
    mesh = plsc.VectorSubcoreMesh(core_axis_name="c", subcore_axis_name="s")
    grp = 40
    while k1 % grp or grp % 2:
        grp -= 8
    npairs = grp // 2
    ngrp = k1 // grp

    @functools.partial(
        pl.kernel,
        out_type=jax.ShapeDtypeStruct((NC, n, d), jnp.float32),
        mesh=mesh,
        scratch_types=[
            pltpu.VMEM((grp, B), jnp.int32),
            pltpu.VMEM((grp, B), jnp.int32),
            pltpu.VMEM((B, d), jnp.float32),
            pltpu.VMEM((B, d), jnp.float32),
            pltpu.SemaphoreType.DMA,
            pltpu.SemaphoreType.DMA,
            pltpu.SemaphoreType.DMA,
            pltpu.SemaphoreType.DMA,
            pltpu.VMEM_SHARED((nacc, d), jnp.float32),
        ],
        compiler_params=pltpu.CompilerParams(use_tc_tiling_on_sc=False),
    )
    def prop_kernel(x_hbm, src_hbm, dst_hbm, zz_hbm, out_hbm,
                    sidx_v, didx_v, rows_a, rows_b, sga, sgb, ssa, ssb, acc):
        cid = lax.axis_index("c")
        sid = lax.axis_index("s")
        wid = cid * NS + sid
        pltpu.sync_copy(zz_hbm, rows_a)
        for off, sz in _chunks(rpt, B):
            pltpu.sync_copy(rows_a.at[pl.ds(0, sz), :],
                            acc.at[pl.ds(sid * rpt + off, sz), :])
        plsc.subcore_barrier()
        base = wid * k1

        for g in range(ngrp):
            pltpu.sync_copy(src_hbm.at[pl.ds(base + g * grp, grp)], sidx_v)
            pltpu.sync_copy(dst_hbm.at[pl.ds(base + g * grp, grp)], didx_v)
            pltpu.async_copy(x_hbm.at[sidx_v.at[0]], rows_a, sga)

            def body(i, c):
                @pl.when(i > 0)
                def _():
                    pltpu.make_async_copy(
                        rows_b, acc.at[didx_v.at[2 * i - 1]], ssb).wait()

                pltpu.async_copy(x_hbm.at[sidx_v.at[2 * i + 1]], rows_b, sgb)
                pltpu.make_async_copy(
                    x_hbm.at[sidx_v.at[2 * i]], rows_a, sga).wait()
                pltpu.async_copy(rows_a, acc.at[didx_v.at[2 * i]], ssa,
                                 add=True)
                pltpu.make_async_copy(
                    rows_a, acc.at[didx_v.at[2 * i]], ssa).wait()

                @pl.when(i < npairs - 1)
                def _():
                    pltpu.async_copy(x_hbm.at[sidx_v.at[2 * i + 2]], rows_a,
                                     sga)

                pltpu.make_async_copy(
                    x_hbm.at[sidx_v.at[2 * i + 1]], rows_b, sgb).wait()
                pltpu.async_copy(rows_b, acc.at[didx_v.at[2 * i + 1]], ssb,
                                 add=True)
                return c

            lax.fori_loop(0, npairs, body, 0)
            pltpu.make_async_copy(rows_b, acc.at[didx_v.at[grp - 1]],
                                  ssb).wait()
        plsc.subcore_barrier()

        full_tiles = n // rpt
        rem = n - full_tiles * rpt

        @pl.when(sid < full_tiles)
        def _():
            for off, sz in _chunks(rpt, B):
                pltpu.sync_copy(acc.at[pl.ds(sid * rpt + off, sz), :],
                                rows_a.at[pl.ds(0, sz), :])
                pltpu.sync_copy(rows_a.at[pl.ds(0, sz), :],
                                out_hbm.at[cid, pl.ds(sid * rpt + off, sz), :])

        if rem:
            @pl.when(sid == full_tiles)
            def _():
                for off, sz in _chunks(rem, B):
                    pltpu.sync_copy(acc.at[pl.ds(sid * rpt + off, sz), :],
                                    rows_a.at[pl.ds(0, sz), :])
                    pltpu.sync_copy(rows_a.at[pl.ds(0, sz), :],
                                    out_hbm.at[cid, pl.ds(sid * rpt + off, sz), :])

    return prop_kernel


def _mm1_body(x_ref, dego_ref, w_ref, o_ref):
    no = lax.rsqrt(jnp.maximum(dego_ref[...], 1.0))
    o_ref[...] = jnp.dot(x_ref[...] * no, w_ref[...],
                         preferred_element_type=jnp.float32)


def _mm2_body(p0_ref, p1_ref, degi_ref, dego_ref, b1_ref, w_ref, o_ref):
    ni = lax.rsqrt(jnp.maximum(degi_ref[...], 1.0))
    no = lax.rsqrt(jnp.maximum(dego_ref[...], 1.0))
    h = jax.nn.relu((p0_ref[...] + p1_ref[...]) * ni + b1_ref[...])
    o_ref[...] = jnp.dot(h * no, w_ref[...],
                         preferred_element_type=jnp.float32)


def _fin_body(q0_ref, q1_ref, degi_ref, b2_ref, o_ref):
    ni = lax.rsqrt(jnp.maximum(degi_ref[...], 1.0))
    o_ref[...] = (q0_ref[...] + q1_ref[...]) * ni + b2_ref[...]


def _row_block(bn, bd):
    return pl.BlockSpec((bn, bd), lambda i: (i, 0))


def _full_block(shape):
    return pl.BlockSpec(shape, lambda i: tuple(0 for _ in shape))


def kernel(in_feat, edge_index, W1, b1, W2, b2):
    n, d_in = in_feat.shape
    d_h = W1.shape[1]
    n_cls = W2.shape[1]
    e = edge_index.shape[1]

    k1 = ((_cdiv(e, NW * B) + 7) // 8) * 8
    e_pad = NW * B * k1
    kd = 2 * k1
    rpt = ((_cdiv(n + 1, NS) + 7) // 8) * 8
    nacc = NS * rpt
    assert n % 1000 == 0

    pad = e_pad - e
    src = edge_index[0]
    dst = edge_index[1]
    src2d = jnp.concatenate([src, jnp.zeros((pad,), jnp.int32)]).reshape(-1, B)
    dummy = n + (jnp.arange(pad, dtype=jnp.int32) % 96)
    dst2d = jnp.concatenate([dst, dummy]).reshape(-1, B)
    ev2d = jnp.concatenate([jnp.ones((e,), jnp.float32),
                            jnp.zeros((pad,), jnp.float32)]).reshape(-1, B)
    zd = jnp.zeros((rpt,), jnp.float32)
    zp1 = jnp.zeros((B, d_h), jnp.float32)
    zp2 = jnp.zeros((B, n_cls), jnp.float32)

    deg = _make_degree_kernel(n, kd, nacc, rpt)(src2d, dst2d, ev2d, zd)
    dego = deg[:n].reshape(n, 1)
    degi = deg[n:].reshape(n, 1)

    bn = 1000
    grid = (n // bn,)

    x1 = pl.pallas_call(
        _mm1_body,
        grid=grid,
        in_specs=[_row_block(bn, d_in), _row_block(bn, 1),
                  _full_block((d_in, d_h))],
        out_specs=_row_block(bn, d_h),
        out_shape=jax.ShapeDtypeStruct((n, d_h), jnp.float32),
    )(in_feat, dego, W1)

    parts1 = _make_prop_kernel(n, d_h, k1, nacc, rpt)(x1, src2d, dst2d, zp1)

    x2 = pl.pallas_call(
        _mm2_body,
        grid=grid,
        in_specs=[_row_block(bn, d_h), _row_block(bn, d_h),
                  _row_block(bn, 1), _row_block(bn, 1),
                  _full_block((1, d_h)), _full_block((d_h, n_cls))],
        out_specs=_row_block(bn, n_cls),
        out_shape=jax.ShapeDtypeStruct((n, n_cls), jnp.float32),
    )(parts1[0], parts1[1], degi, dego, b1.reshape(1, d_h), W2)

    parts2 = _make_prop_kernel(n, n_cls, k1, nacc, rpt)(x2, src2d, dst2d, zp2)

    out = pl.pallas_call(
        _fin_body,
        grid=grid,
        in_specs=[_row_block(bn, n_cls), _row_block(bn, n_cls),
                  _row_block(bn, 1), _full_block((1, n_cls))],
        out_specs=_row_block(bn, n_cls),
        out_shape=jax.ShapeDtypeStruct((n, n_cls), jnp.float32),
    )(parts2[0], parts2[1], degi, b2.reshape(1, n_cls))

    return out


# speedup vs baseline: 1.3033x; 1.0952x over previous
import functools

import jax
import jax.numpy as jnp
from jax import lax
from jax.experimental import pallas as pl
from jax.experimental.pallas import tpu as pltpu
from jax.experimental.pallas import tpu_sc as plsc

NC = 2
NS = 16
NW = NC * NS
B = 128


def _cdiv(a, b):
    return (a + b - 1) // b


def _chunks(total, step):
    out = []
    off = 0
    while off < total:
        out.append((off, min(step, total - off)))
        off += step
    return out


@functools.lru_cache(maxsize=None)
def _make_degree_kernel(n, kd, nacc, rpt):
    mesh = plsc.VectorSubcoreMesh(core_axis_name="c", subcore_axis_name="s")

    @functools.partial(
        pl.kernel,
        out_type=jax.ShapeDtypeStruct((NC * n,), jnp.float32),
        mesh=mesh,
        scratch_types=[
            pltpu.VMEM((kd, B), jnp.int32),
            pltpu.VMEM((kd, B), jnp.float32),
            pltpu.VMEM((rpt,), jnp.float32),
            pltpu.VMEM_SHARED((nacc,), jnp.float32),
        ],
        compiler_params=pltpu.CompilerParams(use_tc_tiling_on_sc=False),
    )
    def deg_kernel(src_hbm, dst_hbm, vals_hbm, zz_hbm, out_hbm, idx_v, val_v,
                   zbuf_v, acc):
        cid = lax.axis_index("c")
        sid = lax.axis_index("s")
        pltpu.sync_copy(zz_hbm, zbuf_v)
        pltpu.sync_copy(zbuf_v, acc.at[pl.ds(sid * rpt, rpt)])

        @pl.when(cid == 0)
        def _():
            pltpu.sync_copy(src_hbm.at[pl.ds(sid * kd, kd)], idx_v)

        @pl.when(cid == 1)
        def _():
            pltpu.sync_copy(dst_hbm.at[pl.ds(sid * kd, kd)], idx_v)

        pltpu.sync_copy(vals_hbm.at[pl.ds(sid * kd, kd)], val_v)
        plsc.subcore_barrier()

        def body(j, c):
            pltpu.sync_copy(val_v.at[j], acc.at[idx_v.at[j]], add=True)
            return c

        lax.fori_loop(0, kd, body, 0)
        plsc.subcore_barrier()

        full_tiles = n // rpt
        rem = n - full_tiles * rpt

        @pl.when(sid < full_tiles)
        def _():
            pltpu.sync_copy(acc.at[pl.ds(sid * rpt, rpt)], zbuf_v)
            pltpu.sync_copy(zbuf_v, out_hbm.at[pl.ds(cid * n + sid * rpt, rpt)])

        if rem:
            @pl.when(sid == full_tiles)
            def _():
                pltpu.sync_copy(acc.at[pl.ds(sid * rpt, rem)],
                                zbuf_v.at[pl.ds(0, rem)])
                pltpu.sync_copy(zbuf_v.at[pl.ds(0, rem)],
                                out_hbm.at[pl.ds(cid * n + sid * rpt, rem)])

    return deg_kernel


@functools.lru_cache(maxsize=None)
def _make_prop_kernel(n, d, k1, nacc, rpt, ncores=NC):
    mesh = plsc.VectorSubcoreMesh(core_axis_name="c", subcore_axis_name="s",
                                  num_cores=ncores)
    grp = 40
    while k1 % grp or grp % 2:
        grp -= 8
    npairs = grp // 2
    ngrp = k1 // grp

    @functools.partial(
        pl.kernel,
        out_type=jax.ShapeDtypeStruct((ncores, n, d), jnp.float32),
        mesh=mesh,
        scratch_types=[
            pltpu.VMEM((grp, B), jnp.int32),
            pltpu.VMEM((grp, B), jnp.int32),
            pltpu.VMEM((B, d), jnp.float32),
            pltpu.VMEM((B, d), jnp.float32),
            pltpu.SemaphoreType.DMA,
            pltpu.SemaphoreType.DMA,
            pltpu.SemaphoreType.DMA,
            pltpu.SemaphoreType.DMA,
            pltpu.VMEM_SHARED((nacc, d), jnp.float32),
        ],
        compiler_params=pltpu.CompilerParams(use_tc_tiling_on_sc=False),
    )
    def prop_kernel(x_hbm, src_hbm, dst_hbm, zz_hbm, out_hbm,
                    sidx_v, didx_v, rows_a, rows_b, sga, sgb, ssa, ssb, acc):
        cid = lax.axis_index("c")
        sid = lax.axis_index("s")
        wid = cid * NS + sid
        pltpu.sync_copy(zz_hbm, rows_a)
        for off, sz in _chunks(rpt, B):
            pltpu.sync_copy(rows_a.at[pl.ds(0, sz), :],
                            acc.at[pl.ds(sid * rpt + off, sz), :])
        plsc.subcore_barrier()
        base = wid * k1

        for g in range(ngrp):
            pltpu.sync_copy(src_hbm.at[pl.ds(base + g * grp, grp)], sidx_v)
            pltpu.sync_copy(dst_hbm.at[pl.ds(base + g * grp, grp)], didx_v)
            pltpu.async_copy(x_hbm.at[sidx_v.at[0]], rows_a, sga)

            def body(i, c):
                @pl.when(i > 0)
                def _():
                    pltpu.make_async_copy(
                        rows_b, acc.at[didx_v.at[2 * i - 1]], ssb).wait()

                pltpu.async_copy(x_hbm.at[sidx_v.at[2 * i + 1]], rows_b, sgb)
                pltpu.make_async_copy(
                    x_hbm.at[sidx_v.at[2 * i]], rows_a, sga).wait()
                pltpu.async_copy(rows_a, acc.at[didx_v.at[2 * i]], ssa,
                                 add=True)
                pltpu.make_async_copy(
                    rows_a, acc.at[didx_v.at[2 * i]], ssa).wait()

                @pl.when(i < npairs - 1)
                def _():
                    pltpu.async_copy(x_hbm.at[sidx_v.at[2 * i + 2]], rows_a,
                                     sga)

                pltpu.make_async_copy(
                    x_hbm.at[sidx_v.at[2 * i + 1]], rows_b, sgb).wait()
                pltpu.async_copy(rows_b, acc.at[didx_v.at[2 * i + 1]], ssb,
                                 add=True)
                return c

            lax.fori_loop(0, npairs, body, 0)
            pltpu.make_async_copy(rows_b, acc.at[didx_v.at[grp - 1]],
                                  ssb).wait()
        plsc.subcore_barrier()

        full_tiles = n // rpt
        rem = n - full_tiles * rpt

        @pl.when(sid < full_tiles)
        def _():
            for off, sz in _chunks(rpt, B):
                pltpu.sync_copy(acc.at[pl.ds(sid * rpt + off, sz), :],
                                rows_a.at[pl.ds(0, sz), :])
                pltpu.sync_copy(rows_a.at[pl.ds(0, sz), :],
                                out_hbm.at[cid, pl.ds(sid * rpt + off, sz), :])

        if rem:
            @pl.when(sid == full_tiles)
            def _():
                for off, sz in _chunks(rem, B):
                    pltpu.sync_copy(acc.at[pl.ds(sid * rpt + off, sz), :],
                                    rows_a.at[pl.ds(0, sz), :])
                    pltpu.sync_copy(rows_a.at[pl.ds(0, sz), :],
                                    out_hbm.at[cid, pl.ds(sid * rpt + off, sz), :])

    return prop_kernel


def _mm1_body(x_ref, dego_ref, w_ref, o_ref):
    no = lax.rsqrt(jnp.maximum(dego_ref[...], 1.0))
    o_ref[...] = jnp.dot(x_ref[...] * no, w_ref[...],
                         preferred_element_type=jnp.float32)


def _mm2_body(p0_ref, p1_ref, degi_ref, dego_ref, b1_ref, w_ref, o_ref):
    ni = lax.rsqrt(jnp.maximum(degi_ref[...], 1.0))
    no = lax.rsqrt(jnp.maximum(dego_ref[...], 1.0))
    h = jax.nn.relu((p0_ref[...] + p1_ref[...]) * ni + b1_ref[...])
    o_ref[...] = jnp.dot(h * no, w_ref[...],
                         preferred_element_type=jnp.float32)


def _fin_body(q0_ref, q1_ref, degi_ref, b2_ref, o_ref):
    ni = lax.rsqrt(jnp.maximum(degi_ref[...], 1.0))
    o_ref[...] = (q0_ref[...] + q1_ref[...]) * ni + b2_ref[...]


def _mm2_body1(p0_ref, degi_ref, dego_ref, b1_ref, w_ref, o_ref):
    ni = lax.rsqrt(jnp.maximum(degi_ref[...], 1.0))
    no = lax.rsqrt(jnp.maximum(dego_ref[...], 1.0))
    h = jax.nn.relu(p0_ref[...] * ni + b1_ref[...])
    o_ref[...] = jnp.dot(h * no, w_ref[...],
                         preferred_element_type=jnp.float32)


def _fin_body1(q0_ref, degi_ref, b2_ref, o_ref):
    ni = lax.rsqrt(jnp.maximum(degi_ref[...], 1.0))
    o_ref[...] = q0_ref[...] * ni + b2_ref[...]


def _row_block(bn, bd):
    return pl.BlockSpec((bn, bd), lambda i: (i, 0))


def _full_block(shape):
    return pl.BlockSpec(shape, lambda i: tuple(0 for _ in shape))


def kernel(in_feat, edge_index, W1, b1, W2, b2):
    n, d_in = in_feat.shape
    d_h = W1.shape[1]
    n_cls = W2.shape[1]
    e = edge_index.shape[1]

    pcores = 1
    k1 = ((_cdiv(e, pcores * NS * B) + 7) // 8) * 8
    e_pad = pcores * NS * B * k1
    kd = e_pad // (NS * B)
    rpt = ((_cdiv(n + 1, NS) + 7) // 8) * 8
    nacc = NS * rpt
    assert n % 1000 == 0

    pad = e_pad - e
    src = edge_index[0]
    dst = edge_index[1]
    src2d = jnp.concatenate([src, jnp.zeros((pad,), jnp.int32)]).reshape(-1, B)
    dummy = n + (jnp.arange(pad, dtype=jnp.int32) % 96)
    dst2d = jnp.concatenate([dst, dummy]).reshape(-1, B)
    ev2d = jnp.concatenate([jnp.ones((e,), jnp.float32),
                            jnp.zeros((pad,), jnp.float32)]).reshape(-1, B)
    zd = jnp.zeros((rpt,), jnp.float32)
    zp1 = jnp.zeros((B, d_h), jnp.float32)
    zp2 = jnp.zeros((B, n_cls), jnp.float32)

    deg = _make_degree_kernel(n, kd, nacc, rpt)(src2d, dst2d, ev2d, zd)
    dego = deg[:n].reshape(n, 1)
    degi = deg[n:].reshape(n, 1)

    bn = 1000
    grid = (n // bn,)

    x1 = pl.pallas_call(
        _mm1_body,
        grid=grid,
        in_specs=[_row_block(bn, d_in), _row_block(bn, 1),
                  _full_block((d_in, d_h))],
        out_specs=_row_block(bn, d_h),
        out_shape=jax.ShapeDtypeStruct((n, d_h), jnp.float32),
    )(in_feat, dego, W1)

    parts1 = _make_prop_kernel(n, d_h, k1, nacc, rpt, pcores)(
        x1, src2d, dst2d, zp1)
    agg1 = parts1[0] if pcores == 1 else None

    x2 = pl.pallas_call(
        _mm2_body1,
        grid=grid,
        in_specs=[_row_block(bn, d_h),
                  _row_block(bn, 1), _row_block(bn, 1),
                  _full_block((1, d_h)), _full_block((d_h, n_cls))],
        out_specs=_row_block(bn, n_cls),
        out_shape=jax.ShapeDtypeStruct((n, n_cls), jnp.float32),
    )(agg1, degi, dego, b1.reshape(1, d_h), W2)

    parts2 = _make_prop_kernel(n, n_cls, k1, nacc, rpt, pcores)(
        x2, src2d, dst2d, zp2)

    out = pl.pallas_call(
        _fin_body1,
        grid=grid,
        in_specs=[_row_block(bn, n_cls),
                  _row_block(bn, 1), _full_block((1, n_cls))],
        out_specs=_row_block(bn, n_cls),
        out_shape=jax.ShapeDtypeStruct((n, n_cls), jnp.float32),
    )(parts2[0], degi, b2.reshape(1, n_cls))

    return out


# speedup vs baseline: 1.4982x; 1.1496x over previous
import functools

import jax
import jax.numpy as jnp
from jax import lax
from jax.experimental import pallas as pl
from jax.experimental.pallas import tpu as pltpu
from jax.experimental.pallas import tpu_sc as plsc

NC = 2
NS = 16
NW = NC * NS
B = 128


def _cdiv(a, b):
    return (a + b - 1) // b


def _chunks(total, step):
    out = []
    off = 0
    while off < total:
        out.append((off, min(step, total - off)))
        off += step
    return out


@functools.lru_cache(maxsize=None)
def _make_degree_kernel(n, kd, nacc, rpt):
    mesh = plsc.VectorSubcoreMesh(core_axis_name="c", subcore_axis_name="s")

    @functools.partial(
        pl.kernel,
        out_type=jax.ShapeDtypeStruct((NC * n,), jnp.float32),
        mesh=mesh,
        scratch_types=[
            pltpu.VMEM((kd, B), jnp.int32),
            pltpu.VMEM((kd, B), jnp.float32),
            pltpu.VMEM((rpt,), jnp.float32),
            pltpu.VMEM_SHARED((nacc,), jnp.float32),
        ],
    )
    def deg_kernel(src_hbm, dst_hbm, vals_hbm, zz_hbm, out_hbm, idx_v, val_v,
                   zbuf_v, acc):
        cid = lax.axis_index("c")
        sid = lax.axis_index("s")
        pltpu.sync_copy(zz_hbm, zbuf_v)
        pltpu.sync_copy(zbuf_v, acc.at[pl.ds(sid * rpt, rpt)])

        @pl.when(cid == 0)
        def _():
            pltpu.sync_copy(src_hbm.at[pl.ds(sid * kd, kd)], idx_v)

        @pl.when(cid == 1)
        def _():
            pltpu.sync_copy(dst_hbm.at[pl.ds(sid * kd, kd)], idx_v)

        pltpu.sync_copy(vals_hbm.at[pl.ds(sid * kd, kd)], val_v)
        plsc.subcore_barrier()

        def body(j, c):
            pltpu.sync_copy(val_v.at[j], acc.at[idx_v.at[j]], add=True)
            return c

        lax.fori_loop(0, kd, body, 0)
        plsc.subcore_barrier()

        full_tiles = n // rpt
        rem = n - full_tiles * rpt

        @pl.when(sid < full_tiles)
        def _():
            pltpu.sync_copy(acc.at[pl.ds(sid * rpt, rpt)], zbuf_v)
            pltpu.sync_copy(zbuf_v, out_hbm.at[pl.ds(cid * n + sid * rpt, rpt)])

        if rem:
            @pl.when(sid == full_tiles)
            def _():
                pltpu.sync_copy(acc.at[pl.ds(sid * rpt, rem)],
                                zbuf_v.at[pl.ds(0, rem)])
                pltpu.sync_copy(zbuf_v.at[pl.ds(0, rem)],
                                out_hbm.at[pl.ds(cid * n + sid * rpt, rem)])

    return deg_kernel


@functools.lru_cache(maxsize=None)
def _make_prop_kernel(n, d, k1, nacc, rpt, ncores=NC):
    mesh = plsc.VectorSubcoreMesh(core_axis_name="c", subcore_axis_name="s",
                                  num_cores=ncores)
    grp = 40
    while k1 % grp or grp % 2:
        grp -= 8
    npairs = grp // 2
    ngrp = k1 // grp

    @functools.partial(
        pl.kernel,
        out_type=jax.ShapeDtypeStruct((ncores, n, d), jnp.float32),
        mesh=mesh,
        scratch_types=[
            pltpu.VMEM((grp, B), jnp.int32),
            pltpu.VMEM((grp, B), jnp.int32),
            pltpu.VMEM((B, d), jnp.float32),
            pltpu.VMEM((B, d), jnp.float32),
            pltpu.SemaphoreType.DMA,
            pltpu.SemaphoreType.DMA,
            pltpu.SemaphoreType.DMA,
            pltpu.SemaphoreType.DMA,
            pltpu.VMEM_SHARED((nacc, d), jnp.float32),
        ],
        compiler_params=pltpu.CompilerParams(use_tc_tiling_on_sc=False),
    )
    def prop_kernel(x_hbm, src_hbm, dst_hbm, zz_hbm, out_hbm,
                    sidx_v, didx_v, rows_a, rows_b, sga, sgb, ssa, ssb, acc):
        cid = lax.axis_index("c")
        sid = lax.axis_index("s")
        wid = cid * NS + sid
        pltpu.sync_copy(zz_hbm, rows_a)
        for off, sz in _chunks(rpt, B):
            pltpu.sync_copy(rows_a.at[pl.ds(0, sz), :],
                            acc.at[pl.ds(sid * rpt + off, sz), :])
        plsc.subcore_barrier()
        base = wid * k1

        for g in range(ngrp):
            pltpu.sync_copy(src_hbm.at[pl.ds(base + g * grp, grp)], sidx_v)
            pltpu.sync_copy(dst_hbm.at[pl.ds(base + g * grp, grp)], didx_v)
            pltpu.async_copy(x_hbm.at[sidx_v.at[0]], rows_a, sga)

            def body(i, c):
                @pl.when(i > 0)
                def _():
                    pltpu.make_async_copy(
                        rows_b, acc.at[didx_v.at[2 * i - 1]], ssb).wait()

                pltpu.async_copy(x_hbm.at[sidx_v.at[2 * i + 1]], rows_b, sgb)
                pltpu.make_async_copy(
                    x_hbm.at[sidx_v.at[2 * i]], rows_a, sga).wait()
                pltpu.async_copy(rows_a, acc.at[didx_v.at[2 * i]], ssa,
                                 add=True)
                pltpu.make_async_copy(
                    rows_a, acc.at[didx_v.at[2 * i]], ssa).wait()

                @pl.when(i < npairs - 1)
                def _():
                    pltpu.async_copy(x_hbm.at[sidx_v.at[2 * i + 2]], rows_a,
                                     sga)

                pltpu.make_async_copy(
                    x_hbm.at[sidx_v.at[2 * i + 1]], rows_b, sgb).wait()
                pltpu.async_copy(rows_b, acc.at[didx_v.at[2 * i + 1]], ssb,
                                 add=True)
                return c

            lax.fori_loop(0, npairs, body, 0)
            pltpu.make_async_copy(rows_b, acc.at[didx_v.at[grp - 1]],
                                  ssb).wait()
        plsc.subcore_barrier()

        full_tiles = n // rpt
        rem = n - full_tiles * rpt

        @pl.when(sid < full_tiles)
        def _():
            for off, sz in _chunks(rpt, B):
                pltpu.sync_copy(acc.at[pl.ds(sid * rpt + off, sz), :],
                                rows_a.at[pl.ds(0, sz), :])
                pltpu.sync_copy(rows_a.at[pl.ds(0, sz), :],
                                out_hbm.at[cid, pl.ds(sid * rpt + off, sz), :])

        if rem:
            @pl.when(sid == full_tiles)
            def _():
                for off, sz in _chunks(rem, B):
                    pltpu.sync_copy(acc.at[pl.ds(sid * rpt + off, sz), :],
                                    rows_a.at[pl.ds(0, sz), :])
                    pltpu.sync_copy(rows_a.at[pl.ds(0, sz), :],
                                    out_hbm.at[cid, pl.ds(sid * rpt + off, sz), :])

    return prop_kernel


def _mm1_body(x_ref, dego_ref, w_ref, o_ref):
    no = lax.rsqrt(jnp.maximum(dego_ref[...], 1.0))
    o_ref[...] = jnp.dot(x_ref[...] * no, w_ref[...],
                         preferred_element_type=jnp.float32)


def _mm2_body(p0_ref, p1_ref, degi_ref, dego_ref, b1_ref, w_ref, o_ref):
    ni = lax.rsqrt(jnp.maximum(degi_ref[...], 1.0))
    no = lax.rsqrt(jnp.maximum(dego_ref[...], 1.0))
    h = jax.nn.relu((p0_ref[...] + p1_ref[...]) * ni + b1_ref[...])
    o_ref[...] = jnp.dot(h * no, w_ref[...],
                         preferred_element_type=jnp.float32)


def _fin_body(q0_ref, q1_ref, degi_ref, b2_ref, o_ref):
    ni = lax.rsqrt(jnp.maximum(degi_ref[...], 1.0))
    o_ref[...] = (q0_ref[...] + q1_ref[...]) * ni + b2_ref[...]


def _mm2_body1(p0_ref, degi_ref, dego_ref, b1_ref, w_ref, o_ref):
    ni = lax.rsqrt(jnp.maximum(degi_ref[...], 1.0))
    no = lax.rsqrt(jnp.maximum(dego_ref[...], 1.0))
    h = jax.nn.relu(p0_ref[...] * ni + b1_ref[...])
    o_ref[...] = jnp.dot(h * no, w_ref[...],
                         preferred_element_type=jnp.float32)


def _fin_body1(q0_ref, degi_ref, b2_ref, o_ref):
    ni = lax.rsqrt(jnp.maximum(degi_ref[...], 1.0))
    o_ref[...] = q0_ref[...] * ni + b2_ref[...]


def _row_block(bn, bd):
    return pl.BlockSpec((bn, bd), lambda i: (i, 0))


def _full_block(shape):
    return pl.BlockSpec(shape, lambda i: tuple(0 for _ in shape))


def kernel(in_feat, edge_index, W1, b1, W2, b2):
    n, d_in = in_feat.shape
    d_h = W1.shape[1]
    n_cls = W2.shape[1]
    e = edge_index.shape[1]

    pcores = 2
    k1 = ((_cdiv(e, pcores * NS * B) + 7) // 8) * 8
    e_pad = pcores * NS * B * k1
    kd = e_pad // (NS * B)
    rpt = ((_cdiv(n + 1, NS) + 7) // 8) * 8
    nacc = NS * rpt
    assert n % 1000 == 0

    pad = e_pad - e
    src = edge_index[0]
    dst = edge_index[1]
    src2d = jnp.concatenate([src, jnp.zeros((pad,), jnp.int32)]).reshape(-1, B)
    dummy = n + (jnp.arange(pad, dtype=jnp.int32) % 96)
    dst2d = jnp.concatenate([dst, dummy]).reshape(-1, B)
    ev2d = jnp.concatenate([jnp.ones((e,), jnp.float32),
                            jnp.zeros((pad,), jnp.float32)]).reshape(-1, B)
    zd = jnp.zeros((rpt,), jnp.float32)
    zp1 = jnp.zeros((B, d_h), jnp.float32)
    zp2 = jnp.zeros((B, n_cls), jnp.float32)

    deg = _make_degree_kernel(n, kd, nacc, rpt)(src2d, dst2d, ev2d, zd)
    dego = deg[:n].reshape(n, 1)
    degi = deg[n:].reshape(n, 1)

    bn = 1000
    grid = (n // bn,)

    x1 = pl.pallas_call(
        _mm1_body,
        grid=grid,
        in_specs=[_row_block(bn, d_in), _row_block(bn, 1),
                  _full_block((d_in, d_h))],
        out_specs=_row_block(bn, d_h),
        out_shape=jax.ShapeDtypeStruct((n, d_h), jnp.float32),
    )(in_feat, dego, W1)

    parts1 = _make_prop_kernel(n, d_h, k1, nacc, rpt, pcores)(
        x1, src2d, dst2d, zp1)

    x2 = pl.pallas_call(
        _mm2_body,
        grid=grid,
        in_specs=[_row_block(bn, d_h), _row_block(bn, d_h),
                  _row_block(bn, 1), _row_block(bn, 1),
                  _full_block((1, d_h)), _full_block((d_h, n_cls))],
        out_specs=_row_block(bn, n_cls),
        out_shape=jax.ShapeDtypeStruct((n, n_cls), jnp.float32),
    )(parts1[0], parts1[1], degi, dego, b1.reshape(1, d_h), W2)

    parts2 = _make_prop_kernel(n, n_cls, k1, nacc, rpt, pcores)(
        x2, src2d, dst2d, zp2)

    out = pl.pallas_call(
        _fin_body,
        grid=grid,
        in_specs=[_row_block(bn, n_cls), _row_block(bn, n_cls),
                  _row_block(bn, 1), _full_block((1, n_cls))],
        out_specs=_row_block(bn, n_cls),
        out_shape=jax.ShapeDtypeStruct((n, n_cls), jnp.float32),
    )(parts2[0], parts2[1], degi, b2.reshape(1, n_cls))

    return out
